# baseline (speedup 1.0000x reference)
import functools

import jax
import jax.numpy as jnp
from jax import lax
from jax.experimental import pallas as pl
from jax.experimental.pallas import tpu as pltpu
from jax.experimental.pallas import tpu_sc as plsc

D_MODEL = 768
D_FF = 2048
N_EXP = 8
T = 2048
ROW_BLK = 256
N_BLK = 24
P = N_BLK * ROW_BLK
F_BLK = 512
N_FB = D_FF // F_BLK
NW = 32
PC = P // NW
TPW = T // NW
TCH = 16
TCS = T // TCH


def _route_body(x_ref, gw_ref, gb_ref, tok_ref, ws_ref, pos0_ref, pos1_ref,
                be_ref, w0_ref, w1_ref):
    x = x_ref[...]
    logits = lax.dot_general(x, gw_ref[...], (((1,), (1,)), ((), ())),
                             preferred_element_type=jnp.float32) + gb_ref[...]
    m = jnp.max(logits, axis=-1, keepdims=True)
    ex = jnp.exp(logits - m)
    probs = ex / jnp.sum(ex, axis=-1, keepdims=True)
    iota8 = lax.broadcasted_iota(jnp.int32, (T, N_EXP), 1)
    m1 = jnp.max(probs, axis=-1, keepdims=True)
    i1 = jnp.min(jnp.where(probs == m1, iota8, N_EXP), axis=-1, keepdims=True)
    probs2 = jnp.where(iota8 == i1, -1.0, probs)
    m2 = jnp.max(probs2, axis=-1, keepdims=True)
    i2 = jnp.min(jnp.where(probs2 == m2, iota8, N_EXP), axis=-1, keepdims=True)
    denom = m1 + m2 + 1e-6
    w0_ref[...] = m1 / denom
    w1_ref[...] = m2 / denom

    oh0 = (iota8 == i1).astype(jnp.float32)
    oh1 = (iota8 == i2).astype(jnp.float32)

    def excl_cumsum_rows(a):
        s = a
        sh = 1
        while sh < T:
            s = s + jnp.concatenate(
                [jnp.zeros((sh, N_EXP), jnp.float32), s[:-sh]], axis=0)
            sh *= 2
        return s - a

    def excl_cumsum_lanes(a):
        s = a
        sh = 1
        while sh < N_EXP:
            s = s + jnp.concatenate(
                [jnp.zeros((1, sh), jnp.float32), s[:, :-sh]], axis=1)
            sh *= 2
        return s - a

    pre0 = excl_cumsum_rows(oh0)
    pre1 = excl_cumsum_rows(oh1)
    c0 = jnp.sum(oh0, axis=0, keepdims=True)
    cnt = c0 + jnp.sum(oh1, axis=0, keepdims=True)
    nblk = jnp.ceil(cnt / ROW_BLK)
    blkstart = excl_cumsum_lanes(nblk)
    segstart = blkstart * ROW_BLK

    dest0 = jnp.sum(oh0 * (segstart + pre0), axis=1, keepdims=True)
    dest1 = jnp.sum(oh1 * (segstart + c0 + pre1), axis=1, keepdims=True)
    pos0_ref[...] = dest0.astype(jnp.int32)
    pos1_ref[...] = dest1.astype(jnp.int32)

    b_iota = lax.broadcasted_iota(jnp.int32, (NW, N_EXP), 0).astype(
        jnp.float32)
    be = jnp.sum((b_iota >= blkstart).astype(jnp.float32), axis=1,
                 keepdims=True) - 1.0
    be_ref[...] = jnp.clip(be, 0.0, N_EXP - 1).astype(jnp.int32)

    p_row = lax.broadcasted_iota(jnp.int32, (1, P), 1).astype(jnp.float32)

    def chunk(c, carry):
        ta, wa = carry
        d0 = pos0_ref[pl.ds(c * TCS, TCS), :].astype(jnp.float32)
        d1 = pos1_ref[pl.ds(c * TCS, TCS), :].astype(jnp.float32)
        wc0 = w0_ref[pl.ds(c * TCS, TCS), :]
        wc1 = w1_ref[pl.ds(c * TCS, TCS), :]
        t_col = (jnp.float32(TCS) * jnp.float32(c)
                 + lax.broadcasted_iota(jnp.int32, (TCS, 1), 0).astype(
                     jnp.float32))
        m0 = d0 == p_row
        m1_ = d1 == p_row
        ta = ta + (jnp.sum(jnp.where(m0, t_col, 0.0), axis=0, keepdims=True)
                   + jnp.sum(jnp.where(m1_, t_col, 0.0), axis=0,
                             keepdims=True))
        wa = wa + (jnp.sum(jnp.where(m0, wc0, 0.0), axis=0, keepdims=True)
                   + jnp.sum(jnp.where(m1_, wc1, 0.0), axis=0, keepdims=True))
        return ta, wa

    tok_acc, ws_acc = lax.fori_loop(
        0, TCH, chunk,
        (jnp.zeros((1, P), jnp.float32), jnp.zeros((1, P), jnp.float32)))
    tok_ref[...] = tok_acc.astype(jnp.int32)
    ws_ref[...] = ws_acc


def _route(x, gate_w, gate_b):
    return pl.pallas_call(
        _route_body,
        out_shape=(
            jax.ShapeDtypeStruct((1, P), jnp.int32),
            jax.ShapeDtypeStruct((1, P), jnp.float32),
            jax.ShapeDtypeStruct((T, 1), jnp.int32),
            jax.ShapeDtypeStruct((T, 1), jnp.int32),
            jax.ShapeDtypeStruct((NW, 1), jnp.int32),
            jax.ShapeDtypeStruct((T, 1), jnp.float32),
            jax.ShapeDtypeStruct((T, 1), jnp.float32),
        ),
    )(x, gate_w, gate_b)


_SUB = PC // 2


@functools.cache
def _gather_kernel():
    @functools.partial(
        pl.kernel,
        mesh=plsc.VectorSubcoreMesh(core_axis_name="c",
                                    subcore_axis_name="s"),
        out_type=jax.ShapeDtypeStruct((P, D_MODEL // 2), jnp.int32),
        scratch_types=[
            pltpu.VMEM((PC,), jnp.int32),
            pltpu.VMEM((_SUB, D_MODEL // 2), jnp.int32),
            pltpu.VMEM((_SUB, D_MODEL // 2), jnp.int32),
            pltpu.SemaphoreType.DMA,
            pltpu.SemaphoreType.DMA,
            pltpu.SemaphoreType.DMA,
            pltpu.SemaphoreType.DMA,
        ],
    )
    def _gather(x_hbm, tok_hbm, xs_hbm, idx_v, rows0_v, rows1_v,
                sg0, sg1, ss0, ss1):
        wid = lax.axis_index("s") * 2 + lax.axis_index("c")
        pltpu.sync_copy(tok_hbm.at[wid], idx_v)
        base = wid * PC
        g0 = pltpu.async_copy(x_hbm.at[idx_v.at[pl.ds(0, _SUB)]],
                              rows0_v, sg0)
        g1 = pltpu.async_copy(x_hbm.at[idx_v.at[pl.ds(_SUB, _SUB)]],
                              rows1_v, sg1)
        g0.wait()
        s0 = pltpu.async_copy(rows0_v, xs_hbm.at[pl.ds(base, _SUB)], ss0)
        g1.wait()
        s1 = pltpu.async_copy(rows1_v, xs_hbm.at[pl.ds(base + _SUB, _SUB)],
                              ss1)
        s0.wait()
        s1.wait()

    return _gather


def _ffn_body(be_ref, xs_ref, ws_ref, w1_ref, b1_ref, w2_ref, b2_ref,
              w3_ref, b3_ref, ys_ref):
    fb = pl.program_id(0)
    b = pl.program_id(1)
    xsb = xs_ref[...].astype(jnp.float32)
    xw1 = lax.dot_general(xsb, w1_ref[...], (((1,), (1,)), ((), ())),
                          preferred_element_type=jnp.float32) + b1_ref[...]
    xw3 = lax.dot_general(xsb, w3_ref[...], (((1,), (1,)), ((), ())),
                          preferred_element_type=jnp.float32) + b3_ref[...]
    h = xw1 * lax.logistic(xw1) * xw3
    yp = lax.dot_general(h, w2_ref[...], (((1,), (1,)), ((), ())),
                         preferred_element_type=jnp.float32)
    ws = ws_ref[...]
    row = b * ROW_BLK

    @pl.when(fb == 0)
    def _init():
        ys_ref[pl.ds(row, ROW_BLK), :] = ws * (yp + b2_ref[...])

    @pl.when(fb != 0)
    def _acc():
        ys_ref[pl.ds(row, ROW_BLK), :] += ws * yp


def _ffn(be, xs, ws, w1, b1, w2, b2, w3, b3):
    grid_spec = pltpu.PrefetchScalarGridSpec(
        num_scalar_prefetch=1,
        grid=(N_FB, N_BLK),
        in_specs=[
            pl.BlockSpec((ROW_BLK, D_MODEL), lambda fb, b, be: (b, 0)),
            pl.BlockSpec((ROW_BLK, 1), lambda fb, b, be: (b, 0)),
            pl.BlockSpec((None, F_BLK, D_MODEL),
                         lambda fb, b, be: (be[b], fb, 0)),
            pl.BlockSpec((None, 1, F_BLK), lambda fb, b, be: (be[b], 0, fb)),
            pl.BlockSpec((None, D_MODEL, F_BLK),
                         lambda fb, b, be: (be[b], 0, fb)),
            pl.BlockSpec((None, 1, D_MODEL), lambda fb, b, be: (be[b], 0, 0)),
            pl.BlockSpec((None, F_BLK, D_MODEL),
                         lambda fb, b, be: (be[b], fb, 0)),
            pl.BlockSpec((None, 1, F_BLK), lambda fb, b, be: (be[b], 0, fb)),
        ],
        out_specs=pl.BlockSpec((P, D_MODEL), lambda fb, b, be: (0, 0)),
    )
    return pl.pallas_call(
        _ffn_body,
        grid_spec=grid_spec,
        out_shape=jax.ShapeDtypeStruct((P, D_MODEL), jnp.float32),
    )(be, xs, ws, w1, b1, w2, b2, w3, b3)


@functools.cache
def _combine_kernel():
    @functools.partial(
        pl.kernel,
        mesh=plsc.VectorSubcoreMesh(core_axis_name="c",
                                    subcore_axis_name="s"),
        out_type=jax.ShapeDtypeStruct((T, D_MODEL), jnp.float32),
        scratch_types=[
            pltpu.VMEM((TPW,), jnp.int32),
            pltpu.VMEM((TPW,), jnp.int32),
            pltpu.VMEM((TPW, D_MODEL), jnp.float32),
            pltpu.VMEM((TPW, D_MODEL), jnp.float32),
            pltpu.SemaphoreType.DMA,
            pltpu.SemaphoreType.DMA,
        ],
    )
    def _combine(ys_hbm, pos0_hbm, pos1_hbm, out_hbm, i0_v, i1_v, r0_v, r1_v,
                 sem0, sem1):
        wid = lax.axis_index("s") * 2 + lax.axis_index("c")
        pltpu.sync_copy(pos0_hbm.at[wid], i0_v)
        pltpu.sync_copy(pos1_hbm.at[wid], i1_v)
        cp0 = pltpu.async_copy(ys_hbm.at[i0_v], r0_v, sem0)
        cp1 = pltpu.async_copy(ys_hbm.at[i1_v], r1_v, sem1)
        cp0.wait()
        cp1.wait()

        def body(j, _):
            for v in range(D_MODEL // 16):
                sl = pl.ds(v * 16, 16)
                r0_v[j, sl] = r0_v[j, sl] + r1_v[j, sl]
            return 0

        lax.fori_loop(0, TPW, body, 0)
        pltpu.sync_copy(r0_v, out_hbm.at[pl.ds(wid * TPW, TPW)])

    return _combine


def kernel(x, gate_w, gate_b, w1, b1, w2, b2, w3, b3):
    tok_row, ws_row, pos0, pos1, be, _w0, _w1 = _route(
        x, gate_w, gate_b.reshape(1, N_EXP))
    tok2d = tok_row.reshape(NW, PC)
    ws_col = ws_row.reshape(P, 1)
    x_i32 = lax.bitcast_convert_type(
        x.astype(jnp.bfloat16).reshape(T, D_MODEL // 2, 2), jnp.int32)
    xs_i32 = _gather_kernel()(x_i32, tok2d)
    xs = lax.bitcast_convert_type(xs_i32, jnp.bfloat16).reshape(P, D_MODEL)
    ys = _ffn(be.reshape(NW)[:N_BLK], xs, ws_col,
              w1, b1.reshape(N_EXP, 1, D_FF),
              w2, b2.reshape(N_EXP, 1, D_MODEL),
              w3, b3.reshape(N_EXP, 1, D_FF))
    return _combine_kernel()(ys, pos0.reshape(NW, TPW), pos1.reshape(NW, TPW))


# speedup vs baseline: 1.7951x; 1.7951x over previous
import functools

import jax
import jax.numpy as jnp
from jax import lax
from jax.experimental import pallas as pl
from jax.experimental.pallas import tpu as pltpu
from jax.experimental.pallas import tpu_sc as plsc

D_MODEL = 768
D_FF = 2048
N_EXP = 8
T = 2048
ROW_BLK = 256
N_BLK = 24
P = N_BLK * ROW_BLK
F_BLK = 512
N_FB = D_FF // F_BLK
NW = 32
PC = P // NW
TPW = T // NW
TCH = 16
TCS = T // TCH


def _route_body(x_ref, gw_ref, gb_ref, tok_ref, ws_ref, pos0_ref, pos1_ref,
                be_ref, w0_ref, w1_ref):
    x = x_ref[...]
    logits = lax.dot_general(x, gw_ref[...], (((1,), (1,)), ((), ())),
                             preferred_element_type=jnp.float32) + gb_ref[...]
    m = jnp.max(logits, axis=-1, keepdims=True)
    ex = jnp.exp(logits - m)
    probs = ex / jnp.sum(ex, axis=-1, keepdims=True)
    iota8 = lax.broadcasted_iota(jnp.int32, (T, N_EXP), 1)
    m1 = jnp.max(probs, axis=-1, keepdims=True)
    i1 = jnp.min(jnp.where(probs == m1, iota8, N_EXP), axis=-1, keepdims=True)
    probs2 = jnp.where(iota8 == i1, -1.0, probs)
    m2 = jnp.max(probs2, axis=-1, keepdims=True)
    i2 = jnp.min(jnp.where(probs2 == m2, iota8, N_EXP), axis=-1, keepdims=True)
    denom = m1 + m2 + 1e-6
    w0_ref[...] = m1 / denom
    w1_ref[...] = m2 / denom

    oh0 = (iota8 == i1).astype(jnp.float32)
    oh1 = (iota8 == i2).astype(jnp.float32)

    def excl_cumsum_rows(a):
        s = a
        sh = 1
        while sh < T:
            s = s + jnp.concatenate(
                [jnp.zeros((sh, N_EXP), jnp.float32), s[:-sh]], axis=0)
            sh *= 2
        return s - a

    def excl_cumsum_lanes(a):
        s = a
        sh = 1
        while sh < N_EXP:
            s = s + jnp.concatenate(
                [jnp.zeros((1, sh), jnp.float32), s[:, :-sh]], axis=1)
            sh *= 2
        return s - a

    pre0 = excl_cumsum_rows(oh0)
    pre1 = excl_cumsum_rows(oh1)
    c0 = jnp.sum(oh0, axis=0, keepdims=True)
    cnt = c0 + jnp.sum(oh1, axis=0, keepdims=True)
    nblk = jnp.ceil(cnt / ROW_BLK)
    blkstart = excl_cumsum_lanes(nblk)
    segstart = blkstart * ROW_BLK

    dest0 = jnp.sum(oh0 * (segstart + pre0), axis=1, keepdims=True)
    dest1 = jnp.sum(oh1 * (segstart + c0 + pre1), axis=1, keepdims=True)
    pos0_ref[...] = dest0.astype(jnp.int32)
    pos1_ref[...] = dest1.astype(jnp.int32)

    b_iota = lax.broadcasted_iota(jnp.int32, (NW, N_EXP), 0).astype(
        jnp.float32)
    be = jnp.sum((b_iota >= blkstart).astype(jnp.float32), axis=1,
                 keepdims=True) - 1.0
    be_ref[...] = jnp.clip(be, 0.0, N_EXP - 1).astype(jnp.int32)

    p_row = lax.broadcasted_iota(jnp.int32, (1, P), 1).astype(jnp.float32)

    def chunk(c, carry):
        ta, wa = carry
        d0 = pos0_ref[pl.ds(c * TCS, TCS), :].astype(jnp.float32)
        d1 = pos1_ref[pl.ds(c * TCS, TCS), :].astype(jnp.float32)
        wc0 = w0_ref[pl.ds(c * TCS, TCS), :]
        wc1 = w1_ref[pl.ds(c * TCS, TCS), :]
        t_col = (jnp.float32(TCS) * jnp.float32(c)
                 + lax.broadcasted_iota(jnp.int32, (TCS, 1), 0).astype(
                     jnp.float32))
        m0 = d0 == p_row
        m1_ = d1 == p_row
        ta = ta + (jnp.sum(jnp.where(m0, t_col, 0.0), axis=0, keepdims=True)
                   + jnp.sum(jnp.where(m1_, t_col, 0.0), axis=0,
                             keepdims=True))
        wa = wa + (jnp.sum(jnp.where(m0, wc0, 0.0), axis=0, keepdims=True)
                   + jnp.sum(jnp.where(m1_, wc1, 0.0), axis=0, keepdims=True))
        return ta, wa

    tok_acc, ws_acc = lax.fori_loop(
        0, TCH, chunk,
        (jnp.zeros((1, P), jnp.float32), jnp.zeros((1, P), jnp.float32)))
    tok_ref[...] = tok_acc.astype(jnp.int32)
    ws_ref[...] = ws_acc


def _route(x, gate_w, gate_b):
    return pl.pallas_call(
        _route_body,
        out_shape=(
            jax.ShapeDtypeStruct((1, P), jnp.int32),
            jax.ShapeDtypeStruct((1, P), jnp.float32),
            jax.ShapeDtypeStruct((T, 1), jnp.int32),
            jax.ShapeDtypeStruct((T, 1), jnp.int32),
            jax.ShapeDtypeStruct((NW, 1), jnp.int32),
            jax.ShapeDtypeStruct((T, 1), jnp.float32),
            jax.ShapeDtypeStruct((T, 1), jnp.float32),
        ),
    )(x, gate_w, gate_b)


def _ffn_body(be_ref, tok_ref, ws_ref, x_ref, w1_ref, b1_ref, w2_ref,
              b2_ref, w3_ref, b3_ref, ys_ref, xbf_scr, xs_scr):
    fb = pl.program_id(0)
    b = pl.program_id(1)
    row = b * ROW_BLK

    @pl.when(jnp.logical_and(fb == 0, b == 0))
    def _cast_x():
        xbf_scr[...] = x_ref[...].astype(jnp.bfloat16)

    @pl.when(fb == 0)
    def _build_xs():
        t_row = lax.broadcasted_iota(jnp.int32, (1, T), 1)
        gmat = (tok_ref[...] == t_row).astype(jnp.bfloat16)
        xs_scr[pl.ds(row, ROW_BLK), :] = lax.dot_general(
            gmat, xbf_scr[...], (((1,), (0,)), ((), ())),
            preferred_element_type=jnp.float32).astype(jnp.bfloat16)

    xsb = xs_scr[pl.ds(row, ROW_BLK), :].astype(jnp.float32)
    xw1 = lax.dot_general(xsb, w1_ref[...], (((1,), (1,)), ((), ())),
                          preferred_element_type=jnp.float32) + b1_ref[...]
    xw3 = lax.dot_general(xsb, w3_ref[...], (((1,), (1,)), ((), ())),
                          preferred_element_type=jnp.float32) + b3_ref[...]
    h = xw1 * lax.logistic(xw1) * xw3
    yp = lax.dot_general(h, w2_ref[...], (((1,), (1,)), ((), ())),
                         preferred_element_type=jnp.float32)
    ws = ws_ref[...]

    @pl.when(fb == 0)
    def _init():
        ys_ref[pl.ds(row, ROW_BLK), :] = ws * (yp + b2_ref[...])

    @pl.when(fb != 0)
    def _acc():
        ys_ref[pl.ds(row, ROW_BLK), :] += ws * yp


def _ffn(be, tok, ws, x, w1, b1, w2, b2, w3, b3):
    grid_spec = pltpu.PrefetchScalarGridSpec(
        num_scalar_prefetch=1,
        grid=(N_FB, N_BLK),
        in_specs=[
            pl.BlockSpec((ROW_BLK, 1), lambda fb, b, be: (b, 0)),
            pl.BlockSpec((ROW_BLK, 1), lambda fb, b, be: (b, 0)),
            pl.BlockSpec((T, D_MODEL), lambda fb, b, be: (0, 0)),
            pl.BlockSpec((None, F_BLK, D_MODEL),
                         lambda fb, b, be: (be[b], fb, 0)),
            pl.BlockSpec((None, 1, F_BLK), lambda fb, b, be: (be[b], 0, fb)),
            pl.BlockSpec((None, D_MODEL, F_BLK),
                         lambda fb, b, be: (be[b], 0, fb)),
            pl.BlockSpec((None, 1, D_MODEL), lambda fb, b, be: (be[b], 0, 0)),
            pl.BlockSpec((None, F_BLK, D_MODEL),
                         lambda fb, b, be: (be[b], fb, 0)),
            pl.BlockSpec((None, 1, F_BLK), lambda fb, b, be: (be[b], 0, fb)),
        ],
        out_specs=pl.BlockSpec((P, D_MODEL), lambda fb, b, be: (0, 0)),
        scratch_shapes=[
            pltpu.VMEM((T, D_MODEL), jnp.bfloat16),
            pltpu.VMEM((P, D_MODEL), jnp.bfloat16),
        ],
    )
    return pl.pallas_call(
        _ffn_body,
        grid_spec=grid_spec,
        out_shape=jax.ShapeDtypeStruct((P, D_MODEL), jnp.float32),
    )(be, tok, ws, x, w1, b1, w2, b2, w3, b3)


@functools.cache
def _combine_kernel():
    @functools.partial(
        pl.kernel,
        mesh=plsc.VectorSubcoreMesh(core_axis_name="c",
                                    subcore_axis_name="s"),
        out_type=jax.ShapeDtypeStruct((T, D_MODEL), jnp.float32),
        scratch_types=[
            pltpu.VMEM((TPW,), jnp.int32),
            pltpu.VMEM((TPW,), jnp.int32),
            pltpu.VMEM((TPW, D_MODEL), jnp.float32),
            pltpu.VMEM((TPW, D_MODEL), jnp.float32),
            pltpu.SemaphoreType.DMA,
            pltpu.SemaphoreType.DMA,
        ],
    )
    def _combine(ys_hbm, pos0_hbm, pos1_hbm, out_hbm, i0_v, i1_v, r0_v, r1_v,
                 sem0, sem1):
        wid = lax.axis_index("s") * 2 + lax.axis_index("c")
        pltpu.sync_copy(pos0_hbm.at[wid], i0_v)
        pltpu.sync_copy(pos1_hbm.at[wid], i1_v)
        cp0 = pltpu.async_copy(ys_hbm.at[i0_v], r0_v, sem0)
        cp1 = pltpu.async_copy(ys_hbm.at[i1_v], r1_v, sem1)
        cp0.wait()
        cp1.wait()

        def body(j, _):
            for v in range(D_MODEL // 16):
                sl = pl.ds(v * 16, 16)
                r0_v[j, sl] = r0_v[j, sl] + r1_v[j, sl]
            return 0

        lax.fori_loop(0, TPW, body, 0)
        pltpu.sync_copy(r0_v, out_hbm.at[pl.ds(wid * TPW, TPW)])

    return _combine


def kernel(x, gate_w, gate_b, w1, b1, w2, b2, w3, b3):
    tok_row, ws_row, pos0, pos1, be, _w0, _w1 = _route(
        x, gate_w, gate_b.reshape(1, N_EXP))
    tok_col = tok_row.reshape(P, 1)
    ws_col = ws_row.reshape(P, 1)
    ys = _ffn(be.reshape(NW)[:N_BLK], tok_col, ws_col, x,
              w1, b1.reshape(N_EXP, 1, D_FF),
              w2, b2.reshape(N_EXP, 1, D_MODEL),
              w3, b3.reshape(N_EXP, 1, D_FF))
    return _combine_kernel()(ys, pos0.reshape(NW, TPW), pos1.reshape(NW, TPW))


# speedup vs baseline: 2.2056x; 1.2287x over previous
import functools

import jax
import jax.numpy as jnp
from jax import lax
from jax.experimental import pallas as pl
from jax.experimental.pallas import tpu as pltpu
from jax.experimental.pallas import tpu_sc as plsc

D_MODEL = 768
D_FF = 2048
N_EXP = 8
T = 2048
ROW_BLK = 256
N_BLK = 24
P = N_BLK * ROW_BLK
F_BLK = 1024
N_FB = D_FF // F_BLK
NW = 32
PC = P // NW
TPW = T // NW
TCH = 16
TCS = T // TCH


def _route_body(x_ref, gw_ref, gb_ref, tok_ref, ws_ref, pos0_ref, pos1_ref,
                be_ref, bv_ref, w0_ref, w1_ref):
    x = x_ref[...]
    logits = lax.dot_general(x, gw_ref[...], (((1,), (1,)), ((), ())),
                             preferred_element_type=jnp.float32) + gb_ref[...]
    m = jnp.max(logits, axis=-1, keepdims=True)
    ex = jnp.exp(logits - m)
    probs = ex / jnp.sum(ex, axis=-1, keepdims=True)
    iota8 = lax.broadcasted_iota(jnp.int32, (T, N_EXP), 1)
    m1 = jnp.max(probs, axis=-1, keepdims=True)
    i1 = jnp.min(jnp.where(probs == m1, iota8, N_EXP), axis=-1, keepdims=True)
    probs2 = jnp.where(iota8 == i1, -1.0, probs)
    m2 = jnp.max(probs2, axis=-1, keepdims=True)
    i2 = jnp.min(jnp.where(probs2 == m2, iota8, N_EXP), axis=-1, keepdims=True)
    denom = m1 + m2 + 1e-6
    w0_ref[...] = m1 / denom
    w1_ref[...] = m2 / denom

    oh0 = (iota8 == i1).astype(jnp.float32)
    oh1 = (iota8 == i2).astype(jnp.float32)

    def excl_cumsum_rows(a):
        s = a
        sh = 1
        while sh < T:
            s = s + jnp.concatenate(
                [jnp.zeros((sh, N_EXP), jnp.float32), s[:-sh]], axis=0)
            sh *= 2
        return s - a

    def excl_cumsum_lanes(a):
        s = a
        sh = 1
        while sh < N_EXP:
            s = s + jnp.concatenate(
                [jnp.zeros((1, sh), jnp.float32), s[:, :-sh]], axis=1)
            sh *= 2
        return s - a

    pre0 = excl_cumsum_rows(oh0)
    pre1 = excl_cumsum_rows(oh1)
    c0 = jnp.sum(oh0, axis=0, keepdims=True)
    cnt = c0 + jnp.sum(oh1, axis=0, keepdims=True)
    nblk = jnp.ceil(cnt / ROW_BLK)
    blkstart = excl_cumsum_lanes(nblk)
    segstart = blkstart * ROW_BLK

    dest0 = jnp.sum(oh0 * (segstart + pre0), axis=1, keepdims=True)
    dest1 = jnp.sum(oh1 * (segstart + c0 + pre1), axis=1, keepdims=True)
    pos0_ref[...] = dest0.astype(jnp.int32)
    pos1_ref[...] = dest1.astype(jnp.int32)

    b_iota = lax.broadcasted_iota(jnp.int32, (NW, N_EXP), 0).astype(
        jnp.float32)
    be = jnp.sum((b_iota >= blkstart).astype(jnp.float32), axis=1,
                 keepdims=True) - 1.0
    be_ref[...] = jnp.clip(be, 0.0, N_EXP - 1).astype(jnp.int32)
    totblk = jnp.sum(nblk, axis=1, keepdims=True)
    bv_ref[...] = (b_iota[:, :1] < totblk).astype(jnp.int32)

    p_row = lax.broadcasted_iota(jnp.int32, (1, P), 1).astype(jnp.float32)

    def chunk(c, carry):
        ta, wa = carry
        d0 = pos0_ref[pl.ds(c * TCS, TCS), :].astype(jnp.float32)
        d1 = pos1_ref[pl.ds(c * TCS, TCS), :].astype(jnp.float32)
        wc0 = w0_ref[pl.ds(c * TCS, TCS), :]
        wc1 = w1_ref[pl.ds(c * TCS, TCS), :]
        t_col = (jnp.float32(TCS) * jnp.float32(c)
                 + lax.broadcasted_iota(jnp.int32, (TCS, 1), 0).astype(
                     jnp.float32))
        m0 = d0 == p_row
        m1_ = d1 == p_row
        ta = ta + (jnp.sum(jnp.where(m0, t_col, 0.0), axis=0, keepdims=True)
                   + jnp.sum(jnp.where(m1_, t_col, 0.0), axis=0,
                             keepdims=True))
        wa = wa + (jnp.sum(jnp.where(m0, wc0, 0.0), axis=0, keepdims=True)
                   + jnp.sum(jnp.where(m1_, wc1, 0.0), axis=0, keepdims=True))
        return ta, wa

    tok_acc, ws_acc = lax.fori_loop(
        0, TCH, chunk,
        (jnp.zeros((1, P), jnp.float32), jnp.zeros((1, P), jnp.float32)))
    tok_ref[...] = tok_acc.astype(jnp.int32)
    ws_ref[...] = ws_acc


def _route(x, gate_w, gate_b):
    return pl.pallas_call(
        _route_body,
        out_shape=(
            jax.ShapeDtypeStruct((1, P), jnp.int32),
            jax.ShapeDtypeStruct((1, P), jnp.float32),
            jax.ShapeDtypeStruct((T, 1), jnp.int32),
            jax.ShapeDtypeStruct((T, 1), jnp.int32),
            jax.ShapeDtypeStruct((NW, 1), jnp.int32),
            jax.ShapeDtypeStruct((NW, 1), jnp.int32),
            jax.ShapeDtypeStruct((T, 1), jnp.float32),
            jax.ShapeDtypeStruct((T, 1), jnp.float32),
        ),
    )(x, gate_w, gate_b)


def _ffn_body(be_ref, bv_ref, tok_ref, ws_ref, x_ref, w1_ref, b1_ref,
              w2_ref, b2_ref, w3_ref, b3_ref, ys_ref, xbf_scr, xs_scr):
    fb = pl.program_id(0)
    b = pl.program_id(1)
    row = b * ROW_BLK

    @pl.when(jnp.logical_and(fb == 0, b == 0))
    def _cast_x():
        xbf_scr[...] = x_ref[...].astype(jnp.bfloat16)

    @pl.when(bv_ref[b] == 1)
    def _live():
        @pl.when(fb == 0)
        def _build_xs():
            t_row = lax.broadcasted_iota(jnp.int32, (1, T), 1)
            gmat = (tok_ref[...] == t_row).astype(jnp.bfloat16)
            xs_scr[pl.ds(row, ROW_BLK), :] = lax.dot_general(
                gmat, xbf_scr[...], (((1,), (0,)), ((), ())),
                preferred_element_type=jnp.float32).astype(jnp.bfloat16)

        xsb = xs_scr[pl.ds(row, ROW_BLK), :].astype(jnp.float32)
        xw1 = lax.dot_general(xsb, w1_ref[...], (((1,), (1,)), ((), ())),
                              preferred_element_type=jnp.float32) + b1_ref[...]
        xw3 = lax.dot_general(xsb, w3_ref[...], (((1,), (1,)), ((), ())),
                              preferred_element_type=jnp.float32) + b3_ref[...]
        h = xw1 * lax.logistic(xw1) * xw3
        yp = lax.dot_general(h, w2_ref[...], (((1,), (1,)), ((), ())),
                             preferred_element_type=jnp.float32)
        ws = ws_ref[...]

        @pl.when(fb == 0)
        def _init():
            ys_ref[pl.ds(row, ROW_BLK), :] = ws * (yp + b2_ref[...])

        @pl.when(fb != 0)
        def _acc():
            ys_ref[pl.ds(row, ROW_BLK), :] += ws * yp


def _ffn(be, bv, tok, ws, x, w1, b1, w2, b2, w3, b3):
    grid_spec = pltpu.PrefetchScalarGridSpec(
        num_scalar_prefetch=2,
        grid=(N_FB, N_BLK),
        in_specs=[
            pl.BlockSpec((ROW_BLK, 1), lambda fb, b, be, bv: (b, 0)),
            pl.BlockSpec((ROW_BLK, 1), lambda fb, b, be, bv: (b, 0)),
            pl.BlockSpec((T, D_MODEL), lambda fb, b, be, bv: (0, 0)),
            pl.BlockSpec((None, F_BLK, D_MODEL),
                         lambda fb, b, be, bv: (be[b], fb, 0)),
            pl.BlockSpec((None, 1, F_BLK),
                         lambda fb, b, be, bv: (be[b], 0, fb)),
            pl.BlockSpec((None, D_MODEL, F_BLK),
                         lambda fb, b, be, bv: (be[b], 0, fb)),
            pl.BlockSpec((None, 1, D_MODEL),
                         lambda fb, b, be, bv: (be[b], 0, 0)),
            pl.BlockSpec((None, F_BLK, D_MODEL),
                         lambda fb, b, be, bv: (be[b], fb, 0)),
            pl.BlockSpec((None, 1, F_BLK),
                         lambda fb, b, be, bv: (be[b], 0, fb)),
        ],
        out_specs=pl.BlockSpec((P, D_MODEL), lambda fb, b, be, bv: (0, 0)),
        scratch_shapes=[
            pltpu.VMEM((T, D_MODEL), jnp.bfloat16),
            pltpu.VMEM((P, D_MODEL), jnp.bfloat16),
        ],
    )
    return pl.pallas_call(
        _ffn_body,
        grid_spec=grid_spec,
        out_shape=jax.ShapeDtypeStruct((P, D_MODEL), jnp.float32),
    )(be, bv, tok, ws, x, w1, b1, w2, b2, w3, b3)


@functools.cache
def _combine_kernel():
    @functools.partial(
        pl.kernel,
        mesh=plsc.VectorSubcoreMesh(core_axis_name="c",
                                    subcore_axis_name="s"),
        out_type=jax.ShapeDtypeStruct((T, D_MODEL), jnp.float32),
        scratch_types=[
            pltpu.VMEM((TPW,), jnp.int32),
            pltpu.VMEM((TPW,), jnp.int32),
            pltpu.VMEM((TPW, D_MODEL), jnp.float32),
            pltpu.VMEM((TPW, D_MODEL), jnp.float32),
            pltpu.SemaphoreType.DMA,
            pltpu.SemaphoreType.DMA,
        ],
    )
    def _combine(ys_hbm, pos0_hbm, pos1_hbm, out_hbm, i0_v, i1_v, r0_v, r1_v,
                 sem0, sem1):
        wid = lax.axis_index("s") * 2 + lax.axis_index("c")
        pltpu.sync_copy(pos0_hbm.at[wid], i0_v)
        pltpu.sync_copy(pos1_hbm.at[wid], i1_v)
        cp0 = pltpu.async_copy(ys_hbm.at[i0_v], r0_v, sem0)
        cp1 = pltpu.async_copy(ys_hbm.at[i1_v], r1_v, sem1)
        cp0.wait()
        cp1.wait()

        def body(j, _):
            for v in range(D_MODEL // 16):
                sl = pl.ds(v * 16, 16)
                r0_v[j, sl] = r0_v[j, sl] + r1_v[j, sl]
            return 0

        lax.fori_loop(0, TPW, body, 0)
        pltpu.sync_copy(r0_v, out_hbm.at[pl.ds(wid * TPW, TPW)])

    return _combine


def kernel(x, gate_w, gate_b, w1, b1, w2, b2, w3, b3):
    tok_row, ws_row, pos0, pos1, be, bv, _w0, _w1 = _route(
        x, gate_w, gate_b.reshape(1, N_EXP))
    tok_col = tok_row.reshape(P, 1)
    ws_col = ws_row.reshape(P, 1)
    ys = _ffn(be.reshape(NW)[:N_BLK], bv.reshape(NW)[:N_BLK], tok_col,
              ws_col, x,
              w1, b1.reshape(N_EXP, 1, D_FF),
              w2, b2.reshape(N_EXP, 1, D_MODEL),
              w3, b3.reshape(N_EXP, 1, D_FF))
    return _combine_kernel()(ys, pos0.reshape(NW, TPW), pos1.reshape(NW, TPW))


# speedup vs baseline: 2.3306x; 1.0567x over previous
import functools

import jax
import jax.numpy as jnp
from jax import lax
from jax.experimental import pallas as pl
from jax.experimental.pallas import tpu as pltpu
from jax.experimental.pallas import tpu_sc as plsc

D_MODEL = 768
D_FF = 2048
N_EXP = 8
T = 2048
ROW_BLK = 256
N_BLK = 24
P = N_BLK * ROW_BLK
F_BLK = 1024
N_FB = D_FF // F_BLK
NW = 32
PC = P // NW
TPW = T // NW
TCH = 16
TCS = T // TCH


def _route_body(x_ref, gw_ref, gb_ref, tok_ref, ws_ref, pos0_ref, pos1_ref,
                be_ref, bv_ref, w0_ref, w1_ref):
    x = x_ref[...]
    logits = lax.dot_general(x, gw_ref[...], (((1,), (1,)), ((), ())),
                             preferred_element_type=jnp.float32) + gb_ref[...]
    m = jnp.max(logits, axis=-1, keepdims=True)
    ex = jnp.exp(logits - m)
    probs = ex / jnp.sum(ex, axis=-1, keepdims=True)
    iota8 = lax.broadcasted_iota(jnp.int32, (T, N_EXP), 1)
    m1 = jnp.max(probs, axis=-1, keepdims=True)
    i1 = jnp.min(jnp.where(probs == m1, iota8, N_EXP), axis=-1, keepdims=True)
    probs2 = jnp.where(iota8 == i1, -1.0, probs)
    m2 = jnp.max(probs2, axis=-1, keepdims=True)
    i2 = jnp.min(jnp.where(probs2 == m2, iota8, N_EXP), axis=-1, keepdims=True)
    denom = m1 + m2 + 1e-6
    w0_ref[...] = m1 / denom
    w1_ref[...] = m2 / denom

    oh0 = (iota8 == i1).astype(jnp.float32)
    oh1 = (iota8 == i2).astype(jnp.float32)

    def excl_cumsum_rows(a):
        s = a
        sh = 1
        while sh < T:
            s = s + jnp.concatenate(
                [jnp.zeros((sh, N_EXP), jnp.float32), s[:-sh]], axis=0)
            sh *= 2
        return s - a

    def excl_cumsum_lanes(a):
        s = a
        sh = 1
        while sh < N_EXP:
            s = s + jnp.concatenate(
                [jnp.zeros((1, sh), jnp.float32), s[:, :-sh]], axis=1)
            sh *= 2
        return s - a

    pre0 = excl_cumsum_rows(oh0)
    pre1 = excl_cumsum_rows(oh1)
    c0 = jnp.sum(oh0, axis=0, keepdims=True)
    cnt = c0 + jnp.sum(oh1, axis=0, keepdims=True)
    nblk = jnp.ceil(cnt / ROW_BLK)
    blkstart = excl_cumsum_lanes(nblk)
    segstart = blkstart * ROW_BLK

    dest0 = jnp.sum(oh0 * (segstart + pre0), axis=1, keepdims=True)
    dest1 = jnp.sum(oh1 * (segstart + c0 + pre1), axis=1, keepdims=True)
    pos0_ref[...] = dest0.astype(jnp.int32)
    pos1_ref[...] = dest1.astype(jnp.int32)

    b_iota = lax.broadcasted_iota(jnp.int32, (NW, N_EXP), 0).astype(
        jnp.float32)
    be = jnp.sum((b_iota >= blkstart).astype(jnp.float32), axis=1,
                 keepdims=True) - 1.0
    be_ref[...] = jnp.clip(be, 0.0, N_EXP - 1).astype(jnp.int32)
    totblk = jnp.sum(nblk, axis=1, keepdims=True)
    bv_ref[...] = (b_iota[:, :1] < totblk).astype(jnp.int32)

    p_row = lax.broadcasted_iota(jnp.int32, (1, P), 1).astype(jnp.float32)

    def chunk(c, carry):
        ta, wa = carry
        d0 = pos0_ref[pl.ds(c * TCS, TCS), :].astype(jnp.float32)
        d1 = pos1_ref[pl.ds(c * TCS, TCS), :].astype(jnp.float32)
        wc0 = w0_ref[pl.ds(c * TCS, TCS), :]
        wc1 = w1_ref[pl.ds(c * TCS, TCS), :]
        t_col = (jnp.float32(TCS) * jnp.float32(c)
                 + lax.broadcasted_iota(jnp.int32, (TCS, 1), 0).astype(
                     jnp.float32))
        m0 = d0 == p_row
        m1_ = d1 == p_row
        ta = ta + (jnp.sum(jnp.where(m0, t_col, 0.0), axis=0, keepdims=True)
                   + jnp.sum(jnp.where(m1_, t_col, 0.0), axis=0,
                             keepdims=True))
        wa = wa + (jnp.sum(jnp.where(m0, wc0, 0.0), axis=0, keepdims=True)
                   + jnp.sum(jnp.where(m1_, wc1, 0.0), axis=0, keepdims=True))
        return ta, wa

    tok_acc, ws_acc = lax.fori_loop(
        0, TCH, chunk,
        (jnp.zeros((1, P), jnp.float32), jnp.zeros((1, P), jnp.float32)))
    tok_ref[...] = tok_acc.astype(jnp.int32)
    ws_ref[...] = ws_acc


def _route(x, gate_w, gate_b):
    return pl.pallas_call(
        _route_body,
        out_shape=(
            jax.ShapeDtypeStruct((1, P), jnp.int32),
            jax.ShapeDtypeStruct((1, P), jnp.float32),
            jax.ShapeDtypeStruct((T, 1), jnp.int32),
            jax.ShapeDtypeStruct((T, 1), jnp.int32),
            jax.ShapeDtypeStruct((NW, 1), jnp.int32),
            jax.ShapeDtypeStruct((NW, 1), jnp.int32),
            jax.ShapeDtypeStruct((T, 1), jnp.float32),
            jax.ShapeDtypeStruct((T, 1), jnp.float32),
        ),
    )(x, gate_w, gate_b)


def _ffn_body(be_ref, bv_ref, tok_ref, ws_ref, x_ref, w1_ref, b1_ref,
              w2_ref, b2_ref, w3_ref, b3_ref, ys_ref, xbf_scr):
    b = pl.program_id(0)

    @pl.when(b == 0)
    def _cast_x():
        xbf_scr[...] = x_ref[...].astype(jnp.bfloat16)

    @pl.when(bv_ref[b] == 1)
    def _live():
        t_row = lax.broadcasted_iota(jnp.int32, (1, T), 1)
        gmat = (tok_ref[...] == t_row).astype(jnp.bfloat16)
        xsb = lax.dot_general(gmat, xbf_scr[...], (((1,), (0,)), ((), ())),
                              preferred_element_type=jnp.float32)
        xw1 = lax.dot_general(xsb, w1_ref[...], (((1,), (1,)), ((), ())),
                              preferred_element_type=jnp.float32) + b1_ref[...]
        xw3 = lax.dot_general(xsb, w3_ref[...], (((1,), (1,)), ((), ())),
                              preferred_element_type=jnp.float32) + b3_ref[...]
        h = xw1 * lax.logistic(xw1) * xw3
        yp = lax.dot_general(h, w2_ref[...], (((1,), (1,)), ((), ())),
                             preferred_element_type=jnp.float32)
        ys_ref[...] = ws_ref[...] * (yp + b2_ref[...])


def _ffn(be, bv, tok, ws, x, w1, b1, w2, b2, w3, b3):
    grid_spec = pltpu.PrefetchScalarGridSpec(
        num_scalar_prefetch=2,
        grid=(N_BLK,),
        in_specs=[
            pl.BlockSpec((ROW_BLK, 1), lambda b, be, bv: (b, 0)),
            pl.BlockSpec((ROW_BLK, 1), lambda b, be, bv: (b, 0)),
            pl.BlockSpec((T, D_MODEL), lambda b, be, bv: (0, 0)),
            pl.BlockSpec((None, D_FF, D_MODEL),
                         lambda b, be, bv: (be[b], 0, 0)),
            pl.BlockSpec((None, 1, D_FF), lambda b, be, bv: (be[b], 0, 0)),
            pl.BlockSpec((None, D_MODEL, D_FF),
                         lambda b, be, bv: (be[b], 0, 0)),
            pl.BlockSpec((None, 1, D_MODEL), lambda b, be, bv: (be[b], 0, 0)),
            pl.BlockSpec((None, D_FF, D_MODEL),
                         lambda b, be, bv: (be[b], 0, 0)),
            pl.BlockSpec((None, 1, D_FF), lambda b, be, bv: (be[b], 0, 0)),
        ],
        out_specs=pl.BlockSpec((ROW_BLK, D_MODEL), lambda b, be, bv: (b, 0)),
        scratch_shapes=[
            pltpu.VMEM((T, D_MODEL), jnp.bfloat16),
        ],
    )
    return pl.pallas_call(
        _ffn_body,
        grid_spec=grid_spec,
        out_shape=jax.ShapeDtypeStruct((P, D_MODEL), jnp.float32),
    )(be, bv, tok, ws, x, w1, b1, w2, b2, w3, b3)


@functools.cache
def _combine_kernel():
    @functools.partial(
        pl.kernel,
        mesh=plsc.VectorSubcoreMesh(core_axis_name="c",
                                    subcore_axis_name="s"),
        out_type=jax.ShapeDtypeStruct((T, D_MODEL), jnp.float32),
        scratch_types=[
            pltpu.VMEM((TPW,), jnp.int32),
            pltpu.VMEM((TPW,), jnp.int32),
            pltpu.VMEM((TPW, D_MODEL), jnp.float32),
            pltpu.VMEM((TPW, D_MODEL), jnp.float32),
            pltpu.SemaphoreType.DMA,
            pltpu.SemaphoreType.DMA,
        ],
    )
    def _combine(ys_hbm, pos0_hbm, pos1_hbm, out_hbm, i0_v, i1_v, r0_v, r1_v,
                 sem0, sem1):
        wid = lax.axis_index("s") * 2 + lax.axis_index("c")
        pltpu.sync_copy(pos0_hbm.at[wid], i0_v)
        pltpu.sync_copy(pos1_hbm.at[wid], i1_v)
        cp0 = pltpu.async_copy(ys_hbm.at[i0_v], r0_v, sem0)
        cp1 = pltpu.async_copy(ys_hbm.at[i1_v], r1_v, sem1)
        cp0.wait()
        cp1.wait()

        def body(j, _):
            for v in range(D_MODEL // 16):
                sl = pl.ds(v * 16, 16)
                r0_v[j, sl] = r0_v[j, sl] + r1_v[j, sl]
            return 0

        lax.fori_loop(0, TPW, body, 0)
        pltpu.sync_copy(r0_v, out_hbm.at[pl.ds(wid * TPW, TPW)])

    return _combine


def kernel(x, gate_w, gate_b, w1, b1, w2, b2, w3, b3):
    tok_row, ws_row, pos0, pos1, be, bv, _w0, _w1 = _route(
        x, gate_w, gate_b.reshape(1, N_EXP))
    tok_col = tok_row.reshape(P, 1)
    ws_col = ws_row.reshape(P, 1)
    ys = _ffn(be.reshape(NW)[:N_BLK], bv.reshape(NW)[:N_BLK], tok_col,
              ws_col, x,
              w1, b1.reshape(N_EXP, 1, D_FF),
              w2, b2.reshape(N_EXP, 1, D_MODEL),
              w3, b3.reshape(N_EXP, 1, D_FF))
    return _combine_kernel()(ys, pos0.reshape(NW, TPW), pos1.reshape(NW, TPW))


# speedup vs baseline: 2.3402x; 1.0041x over previous
import functools

import jax
import jax.numpy as jnp
from jax import lax
from jax.experimental import pallas as pl
from jax.experimental.pallas import tpu as pltpu
from jax.experimental.pallas import tpu_sc as plsc

D_MODEL = 768
D_FF = 2048
N_EXP = 8
T = 2048
ROW_BLK = 256
N_BLK = 23
P = N_BLK * ROW_BLK
F_BLK = 1024
N_FB = D_FF // F_BLK
NW = 32
PC = P // NW
TPW = T // NW
TCH = 16
TCS = T // TCH


def _route_body(x_ref, gw_ref, gb_ref, tok_ref, ws_ref, pos0_ref, pos1_ref,
                be_ref, bv_ref, w0_ref, w1_ref):
    x = x_ref[...]
    logits = lax.dot_general(x, gw_ref[...], (((1,), (1,)), ((), ())),
                             preferred_element_type=jnp.float32) + gb_ref[...]
    m = jnp.max(logits, axis=-1, keepdims=True)
    ex = jnp.exp(logits - m)
    probs = ex / jnp.sum(ex, axis=-1, keepdims=True)
    iota8 = lax.broadcasted_iota(jnp.int32, (T, N_EXP), 1)
    m1 = jnp.max(probs, axis=-1, keepdims=True)
    i1 = jnp.min(jnp.where(probs == m1, iota8, N_EXP), axis=-1, keepdims=True)
    probs2 = jnp.where(iota8 == i1, -1.0, probs)
    m2 = jnp.max(probs2, axis=-1, keepdims=True)
    i2 = jnp.min(jnp.where(probs2 == m2, iota8, N_EXP), axis=-1, keepdims=True)
    denom = m1 + m2 + 1e-6
    w0_ref[...] = m1 / denom
    w1_ref[...] = m2 / denom

    oh0 = (iota8 == i1).astype(jnp.float32)
    oh1 = (iota8 == i2).astype(jnp.float32)

    def excl_cumsum_rows(a):
        s = a
        sh = 1
        while sh < T:
            s = s + jnp.concatenate(
                [jnp.zeros((sh, N_EXP), jnp.float32), s[:-sh]], axis=0)
            sh *= 2
        return s - a

    def excl_cumsum_lanes(a):
        s = a
        sh = 1
        while sh < N_EXP:
            s = s + jnp.concatenate(
                [jnp.zeros((1, sh), jnp.float32), s[:, :-sh]], axis=1)
            sh *= 2
        return s - a

    pre0 = excl_cumsum_rows(oh0)
    pre1 = excl_cumsum_rows(oh1)
    c0 = jnp.sum(oh0, axis=0, keepdims=True)
    cnt = c0 + jnp.sum(oh1, axis=0, keepdims=True)
    nblk = jnp.ceil(cnt / ROW_BLK)
    blkstart = excl_cumsum_lanes(nblk)
    segstart = blkstart * ROW_BLK

    dest0 = jnp.sum(oh0 * (segstart + pre0), axis=1, keepdims=True)
    dest1 = jnp.sum(oh1 * (segstart + c0 + pre1), axis=1, keepdims=True)
    pos0_ref[...] = dest0.astype(jnp.int32)
    pos1_ref[...] = dest1.astype(jnp.int32)

    b_iota = lax.broadcasted_iota(jnp.int32, (NW, N_EXP), 0).astype(
        jnp.float32)
    be = jnp.sum((b_iota >= blkstart).astype(jnp.float32), axis=1,
                 keepdims=True) - 1.0
    be_ref[...] = jnp.clip(be, 0.0, N_EXP - 1).astype(jnp.int32)
    totblk = jnp.sum(nblk, axis=1, keepdims=True)
    bv_ref[...] = (b_iota[:, :1] < totblk).astype(jnp.int32)

    p_row = lax.broadcasted_iota(jnp.int32, (1, P), 1).astype(jnp.float32)

    def chunk(c, carry):
        ta, wa = carry
        d0 = pos0_ref[pl.ds(c * TCS, TCS), :].astype(jnp.float32)
        d1 = pos1_ref[pl.ds(c * TCS, TCS), :].astype(jnp.float32)
        wc0 = w0_ref[pl.ds(c * TCS, TCS), :]
        wc1 = w1_ref[pl.ds(c * TCS, TCS), :]
        t_col = (jnp.float32(TCS) * jnp.float32(c)
                 + lax.broadcasted_iota(jnp.int32, (TCS, 1), 0).astype(
                     jnp.float32))
        m0 = d0 == p_row
        m1_ = d1 == p_row
        ta = ta + (jnp.sum(jnp.where(m0, t_col, 0.0), axis=0, keepdims=True)
                   + jnp.sum(jnp.where(m1_, t_col, 0.0), axis=0,
                             keepdims=True))
        wa = wa + (jnp.sum(jnp.where(m0, wc0, 0.0), axis=0, keepdims=True)
                   + jnp.sum(jnp.where(m1_, wc1, 0.0), axis=0, keepdims=True))
        return ta, wa

    tok_acc, ws_acc = lax.fori_loop(
        0, TCH, chunk,
        (jnp.zeros((1, P), jnp.float32), jnp.zeros((1, P), jnp.float32)))
    tok_ref[...] = tok_acc.astype(jnp.int32)
    ws_ref[...] = ws_acc


def _route(x, gate_w, gate_b):
    return pl.pallas_call(
        _route_body,
        out_shape=(
            jax.ShapeDtypeStruct((1, P), jnp.int32),
            jax.ShapeDtypeStruct((1, P), jnp.float32),
            jax.ShapeDtypeStruct((T, 1), jnp.int32),
            jax.ShapeDtypeStruct((T, 1), jnp.int32),
            jax.ShapeDtypeStruct((NW, 1), jnp.int32),
            jax.ShapeDtypeStruct((NW, 1), jnp.int32),
            jax.ShapeDtypeStruct((T, 1), jnp.float32),
            jax.ShapeDtypeStruct((T, 1), jnp.float32),
        ),
    )(x, gate_w, gate_b)


def _ffn_body(be_ref, bv_ref, tok_ref, ws_ref, x_ref, w1_ref, b1_ref,
              w2_ref, b2_ref, w3_ref, b3_ref, ys_ref, xbf_scr):
    b = pl.program_id(0)

    @pl.when(b == 0)
    def _cast_x():
        xbf_scr[...] = x_ref[...].astype(jnp.bfloat16)

    @pl.when(bv_ref[b] == 1)
    def _live():
        t_row = lax.broadcasted_iota(jnp.int32, (1, T), 1)
        gmat = (tok_ref[...] == t_row).astype(jnp.bfloat16)
        xsb = lax.dot_general(gmat, xbf_scr[...], (((1,), (0,)), ((), ())),
                              preferred_element_type=jnp.float32)
        xw1 = lax.dot_general(xsb, w1_ref[...], (((1,), (1,)), ((), ())),
                              preferred_element_type=jnp.float32) + b1_ref[...]
        xw3 = lax.dot_general(xsb, w3_ref[...], (((1,), (1,)), ((), ())),
                              preferred_element_type=jnp.float32) + b3_ref[...]
        h = xw1 * lax.logistic(xw1) * xw3
        yp = lax.dot_general(h, w2_ref[...], (((1,), (1,)), ((), ())),
                             preferred_element_type=jnp.float32)
        ys_ref[...] = ws_ref[...] * (yp + b2_ref[...])


def _ffn(be, bv, tok, ws, x, w1, b1, w2, b2, w3, b3):
    grid_spec = pltpu.PrefetchScalarGridSpec(
        num_scalar_prefetch=2,
        grid=(N_BLK,),
        in_specs=[
            pl.BlockSpec((ROW_BLK, 1), lambda b, be, bv: (b, 0)),
            pl.BlockSpec((ROW_BLK, 1), lambda b, be, bv: (b, 0)),
            pl.BlockSpec((T, D_MODEL), lambda b, be, bv: (0, 0)),
            pl.BlockSpec((None, D_FF, D_MODEL),
                         lambda b, be, bv: (be[b], 0, 0)),
            pl.BlockSpec((None, 1, D_FF), lambda b, be, bv: (be[b], 0, 0)),
            pl.BlockSpec((None, D_MODEL, D_FF),
                         lambda b, be, bv: (be[b], 0, 0)),
            pl.BlockSpec((None, 1, D_MODEL), lambda b, be, bv: (be[b], 0, 0)),
            pl.BlockSpec((None, D_FF, D_MODEL),
                         lambda b, be, bv: (be[b], 0, 0)),
            pl.BlockSpec((None, 1, D_FF), lambda b, be, bv: (be[b], 0, 0)),
        ],
        out_specs=pl.BlockSpec((ROW_BLK, D_MODEL), lambda b, be, bv: (b, 0)),
        scratch_shapes=[
            pltpu.VMEM((T, D_MODEL), jnp.bfloat16),
        ],
    )
    return pl.pallas_call(
        _ffn_body,
        grid_spec=grid_spec,
        out_shape=jax.ShapeDtypeStruct((P, D_MODEL), jnp.float32),
    )(be, bv, tok, ws, x, w1, b1, w2, b2, w3, b3)


@functools.cache
def _combine_kernel():
    @functools.partial(
        pl.kernel,
        mesh=plsc.VectorSubcoreMesh(core_axis_name="c",
                                    subcore_axis_name="s"),
        out_type=jax.ShapeDtypeStruct((T, D_MODEL), jnp.float32),
        scratch_types=[
            pltpu.VMEM((TPW,), jnp.int32),
            pltpu.VMEM((TPW,), jnp.int32),
            pltpu.VMEM((TPW, D_MODEL), jnp.float32),
            pltpu.VMEM((TPW, D_MODEL), jnp.float32),
            pltpu.SemaphoreType.DMA,
            pltpu.SemaphoreType.DMA,
        ],
    )
    def _combine(ys_hbm, pos0_hbm, pos1_hbm, out_hbm, i0_v, i1_v, r0_v, r1_v,
                 sem0, sem1):
        wid = lax.axis_index("s") * 2 + lax.axis_index("c")
        pltpu.sync_copy(pos0_hbm.at[wid], i0_v)
        pltpu.sync_copy(pos1_hbm.at[wid], i1_v)
        cp0 = pltpu.async_copy(ys_hbm.at[i0_v], r0_v, sem0)
        cp1 = pltpu.async_copy(ys_hbm.at[i1_v], r1_v, sem1)
        cp0.wait()
        cp1.wait()

        def body(j, _):
            for v in range(D_MODEL // 16):
                sl = pl.ds(v * 16, 16)
                r0_v[j, sl] = r0_v[j, sl] + r1_v[j, sl]
            return 0

        lax.fori_loop(0, TPW, body, 0)
        pltpu.sync_copy(r0_v, out_hbm.at[pl.ds(wid * TPW, TPW)])

    return _combine


def kernel(x, gate_w, gate_b, w1, b1, w2, b2, w3, b3):
    tok_row, ws_row, pos0, pos1, be, bv, _w0, _w1 = _route(
        x, gate_w, gate_b.reshape(1, N_EXP))
    tok_col = tok_row.reshape(P, 1)
    ws_col = ws_row.reshape(P, 1)
    ys = _ffn(be.reshape(NW)[:N_BLK], bv.reshape(NW)[:N_BLK], tok_col,
              ws_col, x,
              w1, b1.reshape(N_EXP, 1, D_FF),
              w2, b2.reshape(N_EXP, 1, D_MODEL),
              w3, b3.reshape(N_EXP, 1, D_FF))
    return _combine_kernel()(ys, pos0.reshape(NW, TPW), pos1.reshape(NW, TPW))


# speedup vs baseline: 2.3411x; 1.0004x over previous
import functools

import jax
import jax.numpy as jnp
from jax import lax
from jax.experimental import pallas as pl
from jax.experimental.pallas import tpu as pltpu
from jax.experimental.pallas import tpu_sc as plsc

D_MODEL = 768
D_FF = 2048
N_EXP = 8
T = 2048
ROW_BLK = 256
N_BLK = 23
P = N_BLK * ROW_BLK
F_BLK = 1024
N_FB = D_FF // F_BLK
NW = 32
PC = P // NW
TPW = T // NW
TCH = 8
TCS = T // TCH


def _route_body(x_ref, gw_ref, gb_ref, tok_ref, ws_ref, pos0_ref, pos1_ref,
                be_ref, bv_ref, w0_ref, w1_ref):
    x = x_ref[...]
    logits = lax.dot_general(x, gw_ref[...], (((1,), (1,)), ((), ())),
                             preferred_element_type=jnp.float32) + gb_ref[...]
    m = jnp.max(logits, axis=-1, keepdims=True)
    ex = jnp.exp(logits - m)
    probs = ex / jnp.sum(ex, axis=-1, keepdims=True)
    iota8 = lax.broadcasted_iota(jnp.int32, (T, N_EXP), 1)
    m1 = jnp.max(probs, axis=-1, keepdims=True)
    i1 = jnp.min(jnp.where(probs == m1, iota8, N_EXP), axis=-1, keepdims=True)
    probs2 = jnp.where(iota8 == i1, -1.0, probs)
    m2 = jnp.max(probs2, axis=-1, keepdims=True)
    i2 = jnp.min(jnp.where(probs2 == m2, iota8, N_EXP), axis=-1, keepdims=True)
    denom = m1 + m2 + 1e-6
    w0_ref[...] = m1 / denom
    w1_ref[...] = m2 / denom

    oh0 = (iota8 == i1).astype(jnp.float32)
    oh1 = (iota8 == i2).astype(jnp.float32)

    def excl_cumsum_rows(a):
        s = a
        sh = 1
        while sh < T:
            s = s + jnp.concatenate(
                [jnp.zeros((sh, N_EXP), jnp.float32), s[:-sh]], axis=0)
            sh *= 2
        return s - a

    def excl_cumsum_lanes(a):
        s = a
        sh = 1
        while sh < N_EXP:
            s = s + jnp.concatenate(
                [jnp.zeros((1, sh), jnp.float32), s[:, :-sh]], axis=1)
            sh *= 2
        return s - a

    pre0 = excl_cumsum_rows(oh0)
    pre1 = excl_cumsum_rows(oh1)
    c0 = jnp.sum(oh0, axis=0, keepdims=True)
    cnt = c0 + jnp.sum(oh1, axis=0, keepdims=True)
    nblk = jnp.ceil(cnt / ROW_BLK)
    blkstart = excl_cumsum_lanes(nblk)
    segstart = blkstart * ROW_BLK

    dest0 = jnp.sum(oh0 * (segstart + pre0), axis=1, keepdims=True)
    dest1 = jnp.sum(oh1 * (segstart + c0 + pre1), axis=1, keepdims=True)
    pos0_ref[...] = dest0.astype(jnp.int32)
    pos1_ref[...] = dest1.astype(jnp.int32)

    b_iota = lax.broadcasted_iota(jnp.int32, (NW, N_EXP), 0).astype(
        jnp.float32)
    be = jnp.sum((b_iota >= blkstart).astype(jnp.float32), axis=1,
                 keepdims=True) - 1.0
    be_ref[...] = jnp.clip(be, 0.0, N_EXP - 1).astype(jnp.int32)
    totblk = jnp.sum(nblk, axis=1, keepdims=True)
    bv_ref[...] = (b_iota[:, :1] < totblk).astype(jnp.int32)

    p_row = lax.broadcasted_iota(jnp.int32, (1, P), 1).astype(jnp.float32)

    def chunk(c, carry):
        ta, wa = carry
        d0 = pos0_ref[pl.ds(c * TCS, TCS), :].astype(jnp.float32)
        d1 = pos1_ref[pl.ds(c * TCS, TCS), :].astype(jnp.float32)
        wc0 = w0_ref[pl.ds(c * TCS, TCS), :]
        wc1 = w1_ref[pl.ds(c * TCS, TCS), :]
        t_col = (jnp.float32(TCS) * jnp.float32(c)
                 + lax.broadcasted_iota(jnp.int32, (TCS, 1), 0).astype(
                     jnp.float32))
        m0 = d0 == p_row
        m1_ = d1 == p_row
        ta = ta + (jnp.sum(jnp.where(m0, t_col, 0.0), axis=0, keepdims=True)
                   + jnp.sum(jnp.where(m1_, t_col, 0.0), axis=0,
                             keepdims=True))
        wa = wa + (jnp.sum(jnp.where(m0, wc0, 0.0), axis=0, keepdims=True)
                   + jnp.sum(jnp.where(m1_, wc1, 0.0), axis=0, keepdims=True))
        return ta, wa

    tok_acc, ws_acc = lax.fori_loop(
        0, TCH, chunk,
        (jnp.zeros((1, P), jnp.float32), jnp.zeros((1, P), jnp.float32)))
    tok_ref[...] = tok_acc.astype(jnp.int32)
    ws_ref[...] = ws_acc


def _route(x, gate_w, gate_b):
    return pl.pallas_call(
        _route_body,
        out_shape=(
            jax.ShapeDtypeStruct((1, P), jnp.int32),
            jax.ShapeDtypeStruct((1, P), jnp.float32),
            jax.ShapeDtypeStruct((T, 1), jnp.int32),
            jax.ShapeDtypeStruct((T, 1), jnp.int32),
            jax.ShapeDtypeStruct((NW, 1), jnp.int32),
            jax.ShapeDtypeStruct((NW, 1), jnp.int32),
            jax.ShapeDtypeStruct((T, 1), jnp.float32),
            jax.ShapeDtypeStruct((T, 1), jnp.float32),
        ),
    )(x, gate_w, gate_b)


def _ffn_body(be_ref, bv_ref, tok_ref, ws_ref, x_ref, w1_ref, b1_ref,
              w2_ref, b2_ref, w3_ref, b3_ref, ys_ref, xbf_scr):
    b = pl.program_id(0)

    @pl.when(b == 0)
    def _cast_x():
        xbf_scr[...] = x_ref[...].astype(jnp.bfloat16)

    @pl.when(bv_ref[b] == 1)
    def _live():
        t_row = lax.broadcasted_iota(jnp.int32, (1, T), 1)
        gmat = (tok_ref[...] == t_row).astype(jnp.bfloat16)
        xsb = lax.dot_general(gmat, xbf_scr[...], (((1,), (0,)), ((), ())),
                              preferred_element_type=jnp.float32)
        xw1 = lax.dot_general(xsb, w1_ref[...], (((1,), (1,)), ((), ())),
                              preferred_element_type=jnp.float32) + b1_ref[...]
        xw3 = lax.dot_general(xsb, w3_ref[...], (((1,), (1,)), ((), ())),
                              preferred_element_type=jnp.float32) + b3_ref[...]
        h = xw1 * lax.logistic(xw1) * xw3
        yp = lax.dot_general(h, w2_ref[...], (((1,), (1,)), ((), ())),
                             preferred_element_type=jnp.float32)
        ys_ref[...] = ws_ref[...] * (yp + b2_ref[...])


def _ffn(be, bv, tok, ws, x, w1, b1, w2, b2, w3, b3):
    grid_spec = pltpu.PrefetchScalarGridSpec(
        num_scalar_prefetch=2,
        grid=(N_BLK,),
        in_specs=[
            pl.BlockSpec((ROW_BLK, 1), lambda b, be, bv: (b, 0)),
            pl.BlockSpec((ROW_BLK, 1), lambda b, be, bv: (b, 0)),
            pl.BlockSpec((T, D_MODEL), lambda b, be, bv: (0, 0)),
            pl.BlockSpec((None, D_FF, D_MODEL),
                         lambda b, be, bv: (be[b], 0, 0)),
            pl.BlockSpec((None, 1, D_FF), lambda b, be, bv: (be[b], 0, 0)),
            pl.BlockSpec((None, D_MODEL, D_FF),
                         lambda b, be, bv: (be[b], 0, 0)),
            pl.BlockSpec((None, 1, D_MODEL), lambda b, be, bv: (be[b], 0, 0)),
            pl.BlockSpec((None, D_FF, D_MODEL),
                         lambda b, be, bv: (be[b], 0, 0)),
            pl.BlockSpec((None, 1, D_FF), lambda b, be, bv: (be[b], 0, 0)),
        ],
        out_specs=pl.BlockSpec((ROW_BLK, D_MODEL), lambda b, be, bv: (b, 0)),
        scratch_shapes=[
            pltpu.VMEM((T, D_MODEL), jnp.bfloat16),
        ],
    )
    return pl.pallas_call(
        _ffn_body,
        grid_spec=grid_spec,
        out_shape=jax.ShapeDtypeStruct((P, D_MODEL), jnp.float32),
    )(be, bv, tok, ws, x, w1, b1, w2, b2, w3, b3)


@functools.cache
def _combine_kernel():
    @functools.partial(
        pl.kernel,
        mesh=plsc.VectorSubcoreMesh(core_axis_name="c",
                                    subcore_axis_name="s"),
        out_type=jax.ShapeDtypeStruct((T, D_MODEL), jnp.float32),
        scratch_types=[
            pltpu.VMEM((TPW,), jnp.int32),
            pltpu.VMEM((TPW,), jnp.int32),
            pltpu.VMEM((TPW, D_MODEL), jnp.float32),
            pltpu.VMEM((TPW, D_MODEL), jnp.float32),
            pltpu.SemaphoreType.DMA,
            pltpu.SemaphoreType.DMA,
        ],
    )
    def _combine(ys_hbm, pos0_hbm, pos1_hbm, out_hbm, i0_v, i1_v, r0_v, r1_v,
                 sem0, sem1):
        wid = lax.axis_index("s") * 2 + lax.axis_index("c")
        pltpu.sync_copy(pos0_hbm.at[wid], i0_v)
        pltpu.sync_copy(pos1_hbm.at[wid], i1_v)
        cp0 = pltpu.async_copy(ys_hbm.at[i0_v], r0_v, sem0)
        cp1 = pltpu.async_copy(ys_hbm.at[i1_v], r1_v, sem1)
        cp0.wait()
        cp1.wait()

        def body(j, _):
            for v in range(D_MODEL // 16):
                sl = pl.ds(v * 16, 16)
                r0_v[j, sl] = r0_v[j, sl] + r1_v[j, sl]
            return 0

        lax.fori_loop(0, TPW, body, 0)
        pltpu.sync_copy(r0_v, out_hbm.at[pl.ds(wid * TPW, TPW)])

    return _combine


def kernel(x, gate_w, gate_b, w1, b1, w2, b2, w3, b3):
    tok_row, ws_row, pos0, pos1, be, bv, _w0, _w1 = _route(
        x, gate_w, gate_b.reshape(1, N_EXP))
    tok_col = tok_row.reshape(P, 1)
    ws_col = ws_row.reshape(P, 1)
    ys = _ffn(be.reshape(NW)[:N_BLK], bv.reshape(NW)[:N_BLK], tok_col,
              ws_col, x,
              w1, b1.reshape(N_EXP, 1, D_FF),
              w2, b2.reshape(N_EXP, 1, D_MODEL),
              w3, b3.reshape(N_EXP, 1, D_FF))
    return _combine_kernel()(ys, pos0.reshape(NW, TPW), pos1.reshape(NW, TPW))
